# baseline (speedup 1.0000x reference)
import dataclasses
import functools

import jax
import jax.numpy as jnp
from jax import lax
from jax.experimental import pallas as pl
from jax.experimental.pallas import tpu as pltpu
from jax.experimental.pallas import tpu_sc as plsc

_D = 1024
_LANES = 16
_NC, _NS = 2, 16
_NW = _NC * _NS
_CHUNK = 16
_RING = 4
_G = _D // (2 * _LANES)


def _sc_add_pe(x2d, seg1d, pe_bf):
    n = x2d.shape[0]
    per_w = n // _NW
    steps = per_w // _CHUNK
    mesh = plsc.VectorSubcoreMesh(core_axis_name="c", subcore_axis_name="s")
    cp = pltpu.CompilerParams()
    if "needs_layout_passes" in pltpu.CompilerParams.__dataclass_fields__:
        cp = dataclasses.replace(cp, needs_layout_passes=False)

    @functools.partial(
        pl.kernel,
        mesh=mesh,
        compiler_params=cp,
        out_type=jax.ShapeDtypeStruct((n, _D), jnp.float32),
        scratch_types=(
            [pltpu.VMEM((per_w,), jnp.int32)]
            + [pltpu.VMEM((_CHUNK, _D // 2), jnp.int32) for _ in range(_RING)]
            + [pltpu.VMEM((_CHUNK, _D), jnp.float32) for _ in range(_RING)]
            + [pltpu.SemaphoreType.DMA for _ in range(3 * _RING)]
        ),
    )
    def k(x_hbm, seg_hbm, pe_hbm, out_hbm, idx_v, *bufs):
        pbufs = bufs[:_RING]
        xvs = bufs[_RING:2 * _RING]
        gss = bufs[2 * _RING:3 * _RING]
        xss = bufs[3 * _RING:4 * _RING]
        oss = bufs[4 * _RING:5 * _RING]

        wid = lax.axis_index("s") * _NC + lax.axis_index("c")
        base = wid * per_w
        iota2 = lax.iota(jnp.int32, _LANES) * 2
        pltpu.sync_copy(seg_hbm.at[pl.ds(base, per_w)], idx_v)

        def gather_desc(c, b):
            return pltpu.make_async_copy(
                pe_hbm.at[idx_v.at[pl.ds(c * _CHUNK, _CHUNK)]], pbufs[b], gss[b])

        def xin_desc(c, b):
            return pltpu.make_async_copy(
                x_hbm.at[pl.ds(base + c * _CHUNK, _CHUNK)], xvs[b], xss[b])

        def out_desc(c, b):
            return pltpu.make_async_copy(
                xvs[b], out_hbm.at[pl.ds(base + c * _CHUNK, _CHUNK)], oss[b])

        def issue_in(c, b):
            gather_desc(c, b).start()
            xin_desc(c, b).start()

        for c0 in range(_RING - 1):
            issue_in(c0, c0)

        @pl.loop(0, steps, step=_RING)
        def _group(c):
            for b in range(_RING):
                cc = c + b
                bprev = (b - 1) % _RING
                gather_desc(cc, b).wait()
                xin_desc(cc, b).wait()

                @pl.when(cc >= 1)
                def _():
                    out_desc(cc - 1, bprev).wait()

                @pl.when(cc + _RING - 1 < steps)
                def _():
                    issue_in(cc + _RING - 1, bprev)

                @plsc.parallel_loop(0, _CHUNK * _G, unroll=8)
                def _group_add(t):
                    r = lax.shift_right_logical(t, 5)
                    colp = pl.multiple_of(
                        lax.shift_left(lax.bitwise_and(t, _G - 1), 4),
                        _LANES)
                    v16 = pbufs[b][r, pl.ds(colp, _LANES)]
                    v32 = plsc.bitcast(v16, jnp.bfloat16)
                    lo, hi = plsc.unpack(
                        v32, format=plsc.PackFormat.INTERLEAVED,
                        preferred_element_type=jnp.float32)
                    row_v = jnp.full((_LANES,), r, jnp.int32)
                    col_e = iota2 + lax.shift_left(
                        lax.bitwise_and(t, _G - 1), 5)
                    plsc.addupdate_scatter(xvs[b], [row_v, col_e], lo)
                    plsc.addupdate_scatter(xvs[b], [row_v, col_e + 1], hi)

                out_desc(cc, b).start()

        out_desc(steps - 1, (steps - 1) % _RING).wait()

    return k(x2d, seg1d, pe_bf)


def kernel(x, segment, pe):
    b, s, d = x.shape
    v = pe.shape[0]
    pe_bf = pe.reshape(v, d // 2, 2).astype(jnp.bfloat16)
    pe_i32 = jax.lax.bitcast_convert_type(pe_bf, jnp.int32)
    out = _sc_add_pe(x.reshape(b * s, d), segment.reshape(b * s), pe_i32)
    return out.reshape(b, s, d)


# speedup vs baseline: 2.2785x; 2.2785x over previous
import functools

import jax
import jax.numpy as jnp
from jax import lax
from jax.experimental import pallas as pl
from jax.experimental.pallas import tpu as pltpu
from jax.experimental.pallas import tpu_sc as plsc

_D = 1024
_LANES = 16
_NC, _NS = 2, 16
_NW = _NC * _NS
_CHUNK = 8
_RING = 4


def _sc_add_pe(x2d, seg1d, pe):
    n = x2d.shape[0]
    per_w = n // _NW
    steps = per_w // _CHUNK
    mesh = plsc.VectorSubcoreMesh(core_axis_name="c", subcore_axis_name="s")

    @functools.partial(
        pl.kernel,
        mesh=mesh,
        out_type=jax.ShapeDtypeStruct((n, _D), jnp.float32),
        scratch_types=(
            [pltpu.VMEM((per_w,), jnp.int32)]
            + [pltpu.VMEM((_CHUNK, _D), jnp.float32) for _ in range(2 * _RING)]
            + [pltpu.SemaphoreType.DMA for _ in range(3 * _RING)]
        ),
    )
    def k(x_hbm, seg_hbm, pe_hbm, out_hbm, idx_v, *bufs):
        rows = bufs[:_RING]
        xvs = bufs[_RING:2 * _RING]
        gss = bufs[2 * _RING:3 * _RING]
        xss = bufs[3 * _RING:4 * _RING]
        oss = bufs[4 * _RING:5 * _RING]

        wid = lax.axis_index("s") * _NC + lax.axis_index("c")
        base = wid * per_w
        pltpu.sync_copy(seg_hbm.at[pl.ds(base, per_w)], idx_v)

        def gather_desc(c, b):
            return pltpu.make_async_copy(
                pe_hbm.at[idx_v.at[pl.ds(c * _CHUNK, _CHUNK)]], rows[b], gss[b])

        def xin_desc(c, b):
            return pltpu.make_async_copy(
                x_hbm.at[pl.ds(base + c * _CHUNK, _CHUNK)], xvs[b], xss[b])

        def out_desc(c, b):
            return pltpu.make_async_copy(
                rows[b], out_hbm.at[pl.ds(base + c * _CHUNK, _CHUNK)], oss[b])

        def issue_in(c, b):
            gather_desc(c, b).start()
            xin_desc(c, b).start()

        for c0 in range(_RING - 1):
            issue_in(c0, c0)

        @pl.loop(0, steps, step=_RING)
        def _group(c):
            for b in range(_RING):
                cc = c + b
                bprev = (b - 1) % _RING
                gather_desc(cc, b).wait()
                xin_desc(cc, b).wait()

                @pl.when(cc >= 1)
                def _():
                    out_desc(cc - 1, bprev).wait()

                @pl.when(cc + _RING - 1 < steps)
                def _():
                    issue_in(cc + _RING - 1, bprev)

                @plsc.parallel_loop(0, _CHUNK * (_D // _LANES), unroll=16)
                def _pair_add(t):
                    r = lax.shift_right_logical(t, 6)
                    col = pl.multiple_of(
                        lax.shift_left(lax.bitwise_and(t, _D // _LANES - 1), 4),
                        _LANES)
                    sl = pl.ds(col, _LANES)
                    plsc.addupdate(rows[b].at[r, sl], xvs[b][r, sl])

                out_desc(cc, b).start()

        out_desc(steps - 1, (steps - 1) % _RING).wait()

    return k(x2d, seg1d, pe)


def kernel(x, segment, pe):
    b, s, d = x.shape
    out = _sc_add_pe(x.reshape(b * s, d), segment.reshape(b * s), pe)
    return out.reshape(b, s, d)
